# baseline (speedup 1.0000x reference)
import functools

import jax
import jax.numpy as jnp
from jax import lax
from jax.experimental import pallas as pl
from jax.experimental.pallas import tpu as pltpu
from jax.experimental.pallas import tpu_sc as plsc

N_NODES = 10074
N_IN = 64
N_OUT = 10
HID = 128
WID = 80
E = 320000

NPAD = 10240
PADV = NPAD - 1
L = 16
NC = 2
NS = 16
NW = NC * NS
TPR = NPAD // NW
K = 128
E_TOT = E + N_NODES
CAPC = 90
CAPT = CAPC * K
BM = 512
NB = NPAD // BM


def _sc_edge_factory(F):
    mesh = plsc.VectorSubcoreMesh(
        core_axis_name="c", subcore_axis_name="s", num_cores=NC, num_subcores=NS
    )

    @functools.partial(
        pl.kernel,
        out_type=[
            jax.ShapeDtypeStruct((NPAD, F), jnp.float32),
            jax.ShapeDtypeStruct((1, NPAD), jnp.float32),
        ],
        mesh=mesh,
        scratch_types=[
            pltpu.VMEM((NPAD,), jnp.float32),
            pltpu.VMEM((NPAD,), jnp.float32),
            pltpu.VMEM((TPR,), jnp.float32),
            pltpu.VMEM((NW,), jnp.int32),
            pltpu.VMEM((K,), jnp.int32),
            pltpu.VMEM((K,), jnp.int32),
            pltpu.VMEM((K,), jnp.int32),
            pltpu.VMEM((K,), jnp.int32),
            pltpu.VMEM((K,), jnp.float32),
            pltpu.VMEM((K, F), jnp.float32),
            pltpu.VMEM((K, F), jnp.float32),
            pltpu.VMEM((TPR, F), jnp.float32),
            pltpu.SemaphoreType.DMA,
            pltpu.SemaphoreType.DMA,
        ],
        compiler_params=pltpu.CompilerParams(
            needs_layout_passes=False, use_tc_tiling_on_sc=False
        ),
    )
    def sc_edge(h_hbm, esed_hbm, src_hbm, dst_hbm, nch_hbm, acc_hbm, ssum_hbm,
                es_v, ed_v, ssum_l, nch_v, src_a, dst_a, src_b, dst_b, w_c,
                rows_a, rows_b, acc_l, sem_a, sem_b):
        cid = lax.axis_index("c")
        sid = lax.axis_index("s")
        wid = cid * NS + sid

        pltpu.sync_copy(esed_hbm.at[0], es_v)
        pltpu.sync_copy(esed_hbm.at[1], ed_v)
        pltpu.sync_copy(nch_hbm, nch_v)

        zero16 = jnp.zeros((L,), jnp.float32)

        def _zs(i, _):
            ssum_l[pl.ds(i * L, L)] = zero16
            return 0

        lax.fori_loop(0, TPR // L, _zs, 0)

        def _za(j, _):
            for f in range(F // L):
                acc_l[j, pl.ds(f * L, L)] = zero16
            return 0

        lax.fori_loop(0, TPR, _za, 0)

        widv = jnp.zeros((L,), jnp.int32) + wid
        n_t = plsc.load_gather(nch_v, [widv])[0]

        base = wid * CAPT

        def _issue(c, src_c, dst_c, rows, sem):
            off = base + c * K
            pltpu.sync_copy(src_hbm.at[pl.ds(off, K)], src_c)
            pltpu.sync_copy(dst_hbm.at[pl.ds(off, K)], dst_c)
            pltpu.make_async_copy(h_hbm.at[src_c], rows, sem).start()

        def _process(src_c, dst_c, rows, sem):
            pltpu.make_async_copy(h_hbm.at[src_c], rows, sem).wait()

            def _w(j, _):
                sv = src_c[pl.ds(j * L, L)]
                dv = dst_c[pl.ds(j * L, L)]
                e = plsc.load_gather(es_v, [sv]) + plsc.load_gather(ed_v, [dv])
                e = jnp.where(e >= 0, e, 0.2 * e)
                w = jnp.exp(e)
                w_c[pl.ds(j * L, L)] = w
                plsc.addupdate_scatter(ssum_l, [dv], w, mask=sv < N_NODES)
                return 0

            lax.fori_loop(0, K // L, _w, 0)

            cols = [lax.iota(jnp.int32, L) + f * L for f in range(F // L)]
            zi = jnp.zeros((L,), jnp.int32)

            def _scale(j, _):
                wv = w_c[pl.ds(j * L, L)]
                dv = dst_c[pl.ds(j * L, L)]
                for l in range(L):
                    ws = wv[l]
                    rv = zi + dv[l]
                    r = j * L + l
                    for f in range(F // L):
                        plsc.addupdate_scatter(
                            acc_l, [rv, cols[f]], rows[r, pl.ds(f * L, L)] * ws
                        )
                return 0

            lax.fori_loop(0, K // L, _scale, 0)

        @pl.when(n_t > 0)
        def _():
            _issue(0, src_a, dst_a, rows_a, sem_a)

            def _pair(p, _):
                g = 2 * p
                _issue(g + 1, src_b, dst_b, rows_b, sem_b)
                _process(src_a, dst_a, rows_a, sem_a)

                @pl.when(p < n_t // 2 - 1)
                def _():
                    _issue(g + 2, src_a, dst_a, rows_a, sem_a)

                _process(src_b, dst_b, rows_b, sem_b)
                return 0

            lax.fori_loop(0, n_t // 2, _pair, 0)

        pltpu.sync_copy(ssum_l, ssum_hbm.at[0, pl.ds(wid * TPR, TPR)])
        pltpu.sync_copy(acc_l, acc_hbm.at[pl.ds(wid * TPR, TPR)])

    return sc_edge


def _row_mask(i):
    ridx = lax.broadcasted_iota(jnp.int32, (BM, 1), 0) + i * BM
    return ridx < N_NODES


def _tc_project_factory():
    def body(x_ref, wt_ref, as_ref, ad_ref, h_ref, esed_ref):
        i = pl.program_id(0)
        h = jnp.dot(x_ref[...], wt_ref[...], preferred_element_type=jnp.float32)
        h = jnp.where(_row_mask(i), h, 0.0)
        h_ref[...] = h
        esed_ref[0, :] = jnp.sum(h * as_ref[...], axis=1)
        esed_ref[1, :] = jnp.sum(h * ad_ref[...], axis=1)

    return pl.pallas_call(
        body,
        grid=(NB,),
        in_specs=[
            pl.BlockSpec((BM, HID), lambda i: (i, 0)),
            pl.BlockSpec((HID, WID), lambda i: (0, 0)),
            pl.BlockSpec((1, WID), lambda i: (0, 0)),
            pl.BlockSpec((1, WID), lambda i: (0, 0)),
        ],
        out_specs=[
            pl.BlockSpec((BM, WID), lambda i: (i, 0)),
            pl.BlockSpec((2, BM), lambda i: (0, i)),
        ],
        out_shape=[
            jax.ShapeDtypeStruct((NPAD, WID), jnp.float32),
            jax.ShapeDtypeStruct((2, NPAD), jnp.float32),
        ],
    )


def _tc_combine_project_factory(F_in, F_out, leaky_in):
    def body(acc_ref, ss_ref, b_ref, wt_ref, as_ref, ad_ref, h_ref, esed_ref):
        i = pl.program_id(0)
        s = ss_ref[0, :] + 1e-16
        xin = acc_ref[...] / s[:, None] + b_ref[...]
        if leaky_in:
            xin = jnp.where(xin >= 0, xin, 0.1 * xin)
        xin = jnp.where(_row_mask(i), xin, 0.0)
        h = jnp.dot(xin, wt_ref[...], preferred_element_type=jnp.float32)
        h_ref[...] = h
        esed_ref[0, :] = jnp.sum(h * as_ref[...], axis=1)
        esed_ref[1, :] = jnp.sum(h * ad_ref[...], axis=1)

    return pl.pallas_call(
        body,
        grid=(NB,),
        in_specs=[
            pl.BlockSpec((BM, F_in), lambda i: (i, 0)),
            pl.BlockSpec((1, BM), lambda i: (0, i)),
            pl.BlockSpec((1, F_in), lambda i: (0, 0)),
            pl.BlockSpec((F_in, F_out), lambda i: (0, 0)),
            pl.BlockSpec((1, F_out), lambda i: (0, 0)),
            pl.BlockSpec((1, F_out), lambda i: (0, 0)),
        ],
        out_specs=[
            pl.BlockSpec((BM, F_out), lambda i: (i, 0)),
            pl.BlockSpec((2, BM), lambda i: (0, i)),
        ],
        out_shape=[
            jax.ShapeDtypeStruct((NPAD, F_out), jnp.float32),
            jax.ShapeDtypeStruct((2, NPAD), jnp.float32),
        ],
    )


def _tc_combine_skip_factory():
    def body(acc_ref, ss_ref, b_ref, skip_ref, x_ref):
        i = pl.program_id(0)
        s = ss_ref[0, :] + 1e-16
        v = acc_ref[...] / s[:, None] + b_ref[...] + skip_ref[...]
        x_ref[...] = jnp.where(_row_mask(i), v, 0.0)

    return pl.pallas_call(
        body,
        grid=(NB,),
        in_specs=[
            pl.BlockSpec((BM, HID), lambda i: (i, 0)),
            pl.BlockSpec((1, BM), lambda i: (0, i)),
            pl.BlockSpec((1, HID), lambda i: (0, 0)),
            pl.BlockSpec((BM, HID), lambda i: (i, 0)),
        ],
        out_specs=pl.BlockSpec((BM, HID), lambda i: (i, 0)),
        out_shape=jax.ShapeDtypeStruct((NPAD, HID), jnp.float32),
    )


_sc80 = _sc_edge_factory(WID)
_sc128 = _sc_edge_factory(HID)
_tc_project = _tc_project_factory()
_tc_cp_22 = _tc_combine_project_factory(WID, WID, leaky_in=False)
_tc_cp_23 = _tc_combine_project_factory(WID, HID, leaky_in=True)
_tc_skip = _tc_combine_skip_factory()


def kernel(x, n_steps, problem_data_x, problem_data_y, edge_index, W_iv, b_iv,
           W1, a1s, a1d, b1, W2, a2s, a2d, b2, W3, a3s, a3d, b3, W_out, b_out):
    iv = problem_data_x[:, None] @ W_iv.T + b_iv
    x = x.at[N_NODES - N_IN - N_OUT:N_NODES - N_OUT, :4].set(iv)
    xp = jnp.zeros((NPAD, HID), jnp.float32).at[:N_NODES].set(x)

    loops = jnp.arange(N_NODES, dtype=jnp.int32)
    allsrc = jnp.concatenate([edge_index[0], loops])
    alldst = jnp.concatenate([edge_index[1], loops])
    order = jnp.argsort(alldst)
    ssrc = allsrc[order]
    sdst = alldst[order]
    tile_of = sdst // TPR
    bounds = jnp.searchsorted(
        sdst, jnp.arange(NW, dtype=jnp.int32) * TPR).astype(jnp.int32)
    pos = jnp.arange(E_TOT, dtype=jnp.int32) - bounds[tile_of]
    dest = tile_of * CAPT + pos
    src = jnp.full((NW * CAPT,), PADV, jnp.int32).at[dest].set(ssrc)
    dstl = jnp.zeros((NW * CAPT,), jnp.int32).at[dest].set(sdst - tile_of * TPR)
    counts = jnp.diff(jnp.append(bounds, jnp.int32(E_TOT)))
    nch = (-(-counts // (2 * K)) * 2).astype(jnp.int32)

    W1t, W2t, W3t = W1.T, W2.T, W3.T
    a1s2, a1d2 = a1s[None], a1d[None]
    a2s2, a2d2 = a2s[None], a2d[None]
    a3s2, a3d2 = a3s[None], a3d[None]
    b12, b22, b32 = b1[None], b2[None], b3[None]

    def step(_, xc):
        h1, esed1 = _tc_project(xc, W1t, a1s2, a1d2)
        acc1, ss1 = _sc80(h1, esed1, src, dstl, nch)
        h2, esed2 = _tc_cp_22(acc1, ss1, b12, W2t, a2s2, a2d2)
        acc2, ss2 = _sc80(h2, esed2, src, dstl, nch)
        h3, esed3 = _tc_cp_23(acc2, ss2, b22, W3t, a3s2, a3d2)
        acc3, ss3 = _sc128(h3, esed3, src, dstl, nch)
        return _tc_skip(acc3, ss3, b32, xc)

    xf = lax.fori_loop(0, n_steps, step, xp)
    xout = xf[:N_NODES]

    z = (xout[-N_OUT:] @ W_out.T + b_out)[:, 0]
    network_output = jax.nn.softmax(z, axis=-1)
    y = problem_data_y
    loss = jnp.mean(jnp.maximum(network_output, 0.0) - network_output * y
                    + jnp.log1p(jnp.exp(-jnp.abs(network_output))))
    return (xout, loss, network_output, y)


# speedup vs baseline: 4.8569x; 4.8569x over previous
import functools

import jax
import jax.numpy as jnp
from jax import lax
from jax.experimental import pallas as pl
from jax.experimental.pallas import tpu as pltpu
from jax.experimental.pallas import tpu_sc as plsc

N_NODES = 10074
N_IN = 64
N_OUT = 10
HID = 128
WID = 80
E = 320000

NPAD = 10240
PADV = NPAD - 1
L = 16
NC = 2
NS = 16
NW = NC * NS
K = 128
E_TOT = E + N_NODES
CH = -(-E_TOT // (NW * K))
EPAD = CH * NW * K
RPT = NPAD // NS
BM = 512
NB = NPAD // BM


def _sc_edge_factory(F):
    mesh = plsc.VectorSubcoreMesh(
        core_axis_name="c", subcore_axis_name="s", num_cores=NC, num_subcores=NS
    )

    @functools.partial(
        pl.kernel,
        out_type=[
            jax.ShapeDtypeStruct((NC, NPAD, F), jnp.float32),
            jax.ShapeDtypeStruct((NW, NPAD), jnp.float32),
        ],
        mesh=mesh,
        scratch_types=[
            pltpu.VMEM((NPAD,), jnp.float32),
            pltpu.VMEM((NPAD,), jnp.float32),
            pltpu.VMEM((NPAD,), jnp.float32),
            pltpu.VMEM((2, K), jnp.int32),
            pltpu.VMEM((K,), jnp.float32),
            pltpu.VMEM((K, F), jnp.float32),
            pltpu.VMEM_SHARED((NPAD, F), jnp.float32),
            pltpu.SemaphoreType.DMA,
        ],
        compiler_params=pltpu.CompilerParams(
            needs_layout_passes=False, use_tc_tiling_on_sc=False
        ),
    )
    def sc_edge(h_hbm, esed_hbm, sd_hbm, acc_hbm, ssum_hbm,
                es_v, ed_v, ssum_l, sd_c, w_c, rows, acc_sh, sem):
        cid = lax.axis_index("c")
        sid = lax.axis_index("s")
        wid = cid * NS + sid

        pltpu.sync_copy(esed_hbm.at[0], es_v)
        pltpu.sync_copy(esed_hbm.at[1], ed_v)

        zero16 = jnp.zeros((L,), jnp.float32)

        def _zs(i, _):
            ssum_l[pl.ds(i * L, L)] = zero16
            return 0

        lax.fori_loop(0, NPAD // L, _zs, 0)

        def _zr(j, _):
            for f in range(F // L):
                rows[j, pl.ds(f * L, L)] = zero16
            return 0

        lax.fori_loop(0, K, _zr, 0)

        for r in range(RPT // K):
            pltpu.sync_copy(rows, acc_sh.at[pl.ds(sid * RPT + r * K, K)])
        plsc.subcore_barrier()

        base = wid * CH

        def _chunk(c, _):
            pltpu.sync_copy(sd_hbm.at[base + c], sd_c)
            pltpu.async_copy(h_hbm.at[sd_c.at[0]], rows, sem).wait()

            def _w(j, _):
                sv = sd_c[0, pl.ds(j * L, L)]
                dv = sd_c[1, pl.ds(j * L, L)]
                e = plsc.load_gather(es_v, [sv]) + plsc.load_gather(ed_v, [dv])
                e = jnp.where(e >= 0, e, 0.2 * e)
                w = jnp.exp(e)
                w_c[pl.ds(j * L, L)] = w
                plsc.addupdate_scatter(ssum_l, [dv], w)
                return 0

            lax.fori_loop(0, K // L, _w, 0)

            def _scale(j, _):
                wv = w_c[pl.ds(j * L, L)]
                for l in range(L):
                    ws = wv[l]
                    r = j * L + l
                    for f in range(F // L):
                        rows[r, pl.ds(f * L, L)] = rows[r, pl.ds(f * L, L)] * ws
                return 0

            lax.fori_loop(0, K // L, _scale, 0)
            pltpu.sync_copy(rows, acc_sh.at[sd_c.at[1]], add=True)
            return 0

        lax.fori_loop(0, CH, _chunk, 0)
        plsc.subcore_barrier()

        pltpu.sync_copy(ssum_l, ssum_hbm.at[wid])
        for r in range(RPT // K):
            sl = pl.ds(sid * RPT + r * K, K)
            pltpu.sync_copy(acc_sh.at[sl], acc_hbm.at[cid].at[sl])

    return sc_edge


def _tc_project_factory():
    def body(x_ref, wt_ref, as_ref, ad_ref, h_ref, esed_ref):
        h = jnp.dot(x_ref[...], wt_ref[...], preferred_element_type=jnp.float32)
        h_ref[...] = h
        esed_ref[0, :] = jnp.sum(h * as_ref[...], axis=1)
        esed_ref[1, :] = jnp.sum(h * ad_ref[...], axis=1)

    return pl.pallas_call(
        body,
        grid=(NB,),
        in_specs=[
            pl.BlockSpec((BM, HID), lambda i: (i, 0)),
            pl.BlockSpec((HID, WID), lambda i: (0, 0)),
            pl.BlockSpec((1, WID), lambda i: (0, 0)),
            pl.BlockSpec((1, WID), lambda i: (0, 0)),
        ],
        out_specs=[
            pl.BlockSpec((BM, WID), lambda i: (i, 0)),
            pl.BlockSpec((2, BM), lambda i: (0, i)),
        ],
        out_shape=[
            jax.ShapeDtypeStruct((NPAD, WID), jnp.float32),
            jax.ShapeDtypeStruct((2, NPAD), jnp.float32),
        ],
    )


def _tc_combine_project_factory(F_in, F_out, leaky_in):
    def body(acc_a, acc_b, ss_ref, b_ref, wt_ref, as_ref, ad_ref, h_ref, esed_ref):
        a = acc_a[0, :, :] + acc_b[0, :, :]
        s = jnp.sum(ss_ref[...], axis=0) + 1e-16
        xin = a / s[:, None] + b_ref[...]
        if leaky_in:
            xin = jnp.where(xin >= 0, xin, 0.1 * xin)
        h = jnp.dot(xin, wt_ref[...], preferred_element_type=jnp.float32)
        h_ref[...] = h
        esed_ref[0, :] = jnp.sum(h * as_ref[...], axis=1)
        esed_ref[1, :] = jnp.sum(h * ad_ref[...], axis=1)

    return pl.pallas_call(
        body,
        grid=(NB,),
        in_specs=[
            pl.BlockSpec((1, BM, F_in), lambda i: (0, i, 0)),
            pl.BlockSpec((1, BM, F_in), lambda i: (1, i, 0)),
            pl.BlockSpec((NW, BM), lambda i: (0, i)),
            pl.BlockSpec((1, F_in), lambda i: (0, 0)),
            pl.BlockSpec((F_in, F_out), lambda i: (0, 0)),
            pl.BlockSpec((1, F_out), lambda i: (0, 0)),
            pl.BlockSpec((1, F_out), lambda i: (0, 0)),
        ],
        out_specs=[
            pl.BlockSpec((BM, F_out), lambda i: (i, 0)),
            pl.BlockSpec((2, BM), lambda i: (0, i)),
        ],
        out_shape=[
            jax.ShapeDtypeStruct((NPAD, F_out), jnp.float32),
            jax.ShapeDtypeStruct((2, NPAD), jnp.float32),
        ],
    )


def _tc_combine_skip_factory():
    def body(acc_a, acc_b, ss_ref, b_ref, skip_ref, x_ref):
        a = acc_a[0, :, :] + acc_b[0, :, :]
        s = jnp.sum(ss_ref[...], axis=0) + 1e-16
        x_ref[...] = a / s[:, None] + b_ref[...] + skip_ref[...]

    return pl.pallas_call(
        body,
        grid=(NB,),
        in_specs=[
            pl.BlockSpec((1, BM, HID), lambda i: (0, i, 0)),
            pl.BlockSpec((1, BM, HID), lambda i: (1, i, 0)),
            pl.BlockSpec((NW, BM), lambda i: (0, i)),
            pl.BlockSpec((1, HID), lambda i: (0, 0)),
            pl.BlockSpec((BM, HID), lambda i: (i, 0)),
        ],
        out_specs=pl.BlockSpec((BM, HID), lambda i: (i, 0)),
        out_shape=jax.ShapeDtypeStruct((NPAD, HID), jnp.float32),
    )


_sc80 = _sc_edge_factory(WID)
_sc128 = _sc_edge_factory(HID)
_tc_project = _tc_project_factory()
_tc_cp_22 = _tc_combine_project_factory(WID, WID, leaky_in=False)
_tc_cp_23 = _tc_combine_project_factory(WID, HID, leaky_in=True)
_tc_skip = _tc_combine_skip_factory()


def kernel(x, n_steps, problem_data_x, problem_data_y, edge_index, W_iv, b_iv,
           W1, a1s, a1d, b1, W2, a2s, a2d, b2, W3, a3s, a3d, b3, W_out, b_out):
    iv = problem_data_x[:, None] @ W_iv.T + b_iv
    x = x.at[N_NODES - N_IN - N_OUT:N_NODES - N_OUT, :4].set(iv)
    xp = jnp.zeros((NPAD, HID), jnp.float32).at[:N_NODES].set(x)

    loops = jnp.arange(N_NODES, dtype=jnp.int32)
    pad = jnp.full((EPAD - E_TOT,), PADV, jnp.int32)
    src = jnp.concatenate([edge_index[0], loops, pad]).reshape(NW * CH, 1, K)
    dst = jnp.concatenate([edge_index[1], loops, pad]).reshape(NW * CH, 1, K)
    sd = jnp.concatenate([src, dst], axis=1)

    W1t, W2t, W3t = W1.T, W2.T, W3.T
    a1s2, a1d2 = a1s[None], a1d[None]
    a2s2, a2d2 = a2s[None], a2d[None]
    a3s2, a3d2 = a3s[None], a3d[None]
    b12, b22, b32 = b1[None], b2[None], b3[None]

    def step(_, xc):
        h1, esed1 = _tc_project(xc, W1t, a1s2, a1d2)
        acc1, ss1 = _sc80(h1, esed1, sd)
        h2, esed2 = _tc_cp_22(acc1, acc1, ss1, b12, W2t, a2s2, a2d2)
        acc2, ss2 = _sc80(h2, esed2, sd)
        h3, esed3 = _tc_cp_23(acc2, acc2, ss2, b22, W3t, a3s2, a3d2)
        acc3, ss3 = _sc128(h3, esed3, sd)
        return _tc_skip(acc3, acc3, ss3, b32, xc)

    xf = lax.fori_loop(0, n_steps, step, xp)
    xout = xf[:N_NODES]

    z = (xout[-N_OUT:] @ W_out.T + b_out)[:, 0]
    network_output = jax.nn.softmax(z, axis=-1)
    y = problem_data_y
    loss = jnp.mean(jnp.maximum(network_output, 0.0) - network_output * y
                    + jnp.log1p(jnp.exp(-jnp.abs(network_output))))
    return (xout, loss, network_output, y)
